# baseline (speedup 1.0000x reference)
import functools

import jax
import jax.numpy as jnp
from jax import lax
from jax.experimental import pallas as pl
from jax.experimental.pallas import tpu as pltpu
from jax.experimental.pallas import tpu_sc as plsc

D = 64
HALF = 32
N_PAD = 51200
CHUNK = 128
NS = 16
NC = 2
ZR = 128
BR = 512
IB = 15
NRING = 5


def _prep_body(tot_ref, dn_ref, do_ref, s3_ref):
    inv = 1.0 / (jnp.sqrt(dn_ref[...] + do_ref[...]) + 1e-9)
    scaled = inv * tot_ref[...]
    s3_ref[0] = scaled[:, :HALF]
    s3_ref[1] = scaled[:, HALF:]


def _post_body(cw_ref, p3_ref, tot_ref, old_ref, dn_ref, do_ref, out_ref):
    do = do_ref[...]
    inv = 1.0 / (jnp.sqrt(dn_ref[...] + do) + 1e-9)
    w10 = cw_ref[1, 0]
    w11 = cw_ref[1, 1]
    p = jnp.concatenate([p3_ref[0], p3_ref[1]], axis=1)
    last = (w10 * jnp.sqrt(do) * inv) * old_ref[...] + (w11 * inv) * p
    nrm = jnp.maximum(
        jnp.sqrt(jnp.sum(last * last, axis=-1, keepdims=True)), 1e-12)
    out_ref[...] = last / nrm + tot_ref[...]


def _sc_spmm(n_nodes, e_pad):
    e_per_tile = e_pad // NS
    n_chunks = e_per_tile // CHUNK
    nb = n_chunks // IB
    rows_per_tile = N_PAD // NS
    n_zr = rows_per_tile // ZR
    mesh = plsc.VectorSubcoreMesh(core_axis_name="c", subcore_axis_name="s")

    @functools.partial(
        pl.kernel,
        out_type=jax.ShapeDtypeStruct((NC, N_PAD, HALF), jnp.float32),
        mesh=mesh,
        compiler_params=pltpu.CompilerParams(use_tc_tiling_on_sc=False),
        scratch_types=[
            pltpu.VMEM((2, IB * CHUNK), jnp.int32),
            pltpu.VMEM((2, IB, CHUNK), jnp.int32),
            pltpu.VMEM((NRING, CHUNK, HALF), jnp.float32),
            pltpu.VMEM_SHARED((N_PAD, HALF), jnp.float32),
            pltpu.SemaphoreType.DMA,
            pltpu.SemaphoreType.DMA,
            pltpu.SemaphoreType.DMA,
        ],
    )
    def spmm(src_hbm, dst_hbm, s3_hbm, out_hbm,
             sibuf, dibuf, ring, acc, gsem, isem, ssem):
        c = lax.axis_index("c")
        s = lax.axis_index("s")
        s3c = s3_hbm.at[c]
        cbase = s * n_chunks
        ebase = s * e_per_tile

        def zrow(i, carry):
            z = jnp.zeros((16,), jnp.float32)
            ring[0, i, pl.ds(0, 16)] = z
            ring[0, i, pl.ds(16, 16)] = z
            return carry
        lax.fori_loop(0, CHUNK, zrow, 0)
        rbase = s * rows_per_tile

        def zcopy(i, carry):
            pltpu.sync_copy(ring.at[0], acc.at[pl.ds(rbase + i * ZR, ZR)])
            return carry
        lax.fori_loop(0, n_zr, zcopy, 0)
        plsc.subcore_barrier()

        ga = 3
        pltpu.sync_copy(src_hbm.at[pl.ds(ebase, IB * CHUNK)], sibuf.at[0])
        pltpu.sync_copy(dst_hbm.at[pl.ds(cbase, IB)], dibuf.at[0])
        sib0 = sibuf.at[0]
        for k in range(ga):
            pltpu.async_copy(
                s3c.at[sib0.at[pl.ds(k * CHUNK, CHUNK)]], ring.at[k], gsem)

        def block(b, carry):
            pb = lax.rem(b, 2)
            nxt_e = ebase + (b + 1) * IB * CHUNK
            nxt_c = cbase + (b + 1) * IB

            @pl.when(b < nb - 1)
            def _():
                pltpu.async_copy(
                    src_hbm.at[pl.ds(nxt_e, IB * CHUNK)], sibuf.at[1 - pb],
                    isem)
                pltpu.async_copy(
                    dst_hbm.at[pl.ds(nxt_c, IB)], dibuf.at[1 - pb], isem)

            sib = sibuf.at[pb]
            dib = dibuf.at[pb]
            sibn = sibuf.at[1 - pb]
            for k in range(IB):
                slot = k % NRING
                s2 = (k + ga) % NRING
                if k >= 2:
                    pltpu.make_async_copy(
                        ring.at[s2], acc.at[dib.at[k - 2]], ssem).wait()
                else:
                    @pl.when(b > 0)
                    def _():
                        pltpu.make_async_copy(
                            ring.at[s2], acc.at[dib.at[k]], ssem).wait()
                if k < IB - ga:
                    pltpu.async_copy(
                        s3c.at[sib.at[pl.ds((k + ga) * CHUNK, CHUNK)]],
                        ring.at[s2], gsem)
                else:
                    @pl.when(b < nb - 1)
                    def _():
                        if k == IB - ga:
                            pltpu.make_async_copy(
                                src_hbm.at[pl.ds(nxt_e, IB * CHUNK)],
                                sibuf.at[1 - pb], isem).wait()
                            pltpu.make_async_copy(
                                dst_hbm.at[pl.ds(nxt_c, IB)],
                                dibuf.at[1 - pb], isem).wait()
                        pltpu.async_copy(
                            s3c.at[sibn.at[
                                pl.ds((k - (IB - ga)) * CHUNK, CHUNK)]],
                            ring.at[s2], gsem)
                pltpu.make_async_copy(
                    s3c.at[sib.at[pl.ds(k * CHUNK, CHUNK)]], ring.at[slot],
                    gsem).wait()
                pltpu.async_copy(
                    ring.at[slot], acc.at[dib.at[k]], ssem, add=True)
            return carry
        lax.fori_loop(0, nb, block, 0)
        lk = (nb * IB - 2) % NRING
        pltpu.make_async_copy(
            ring.at[lk], acc.at[dibuf.at[0].at[0]], ssem).wait()
        pltpu.make_async_copy(
            ring.at[(lk + 1) % NRING], acc.at[dibuf.at[0].at[1]], ssem).wait()
        plsc.subcore_barrier()

        def wcopy(i, carry):
            r = rbase + i * ZR
            pltpu.sync_copy(acc.at[pl.ds(r, ZR)], ring.at[0])
            pltpu.sync_copy(ring.at[0], out_hbm.at[c].at[pl.ds(r, ZR)])
            return carry
        lax.fori_loop(0, n_zr, wcopy, 0)

    return spmm


def kernel(edge_index, now_user_degree, now_item_degree, old_user_degree,
           old_item_degree, old_emb0, old_emb1, user_table, item_table,
           conv_w):
    n = user_table.shape[0] + item_table.shape[0]
    e = edge_index.shape[1]
    e_pad = -(-e // (NS * CHUNK * IB)) * (NS * CHUNK * IB)

    total = jnp.concatenate([user_table, item_table], axis=0)
    deg_new = jnp.concatenate([now_user_degree, now_item_degree], axis=0)
    deg_old = jnp.concatenate([old_user_degree, old_item_degree], axis=0)
    ei = edge_index.astype(jnp.int32)
    pad_e = e_pad - e
    src = jnp.concatenate([ei[1], jnp.zeros((pad_e,), jnp.int32)])
    dst = jnp.concatenate(
        [ei[0], jnp.full((pad_e,), N_PAD - 1, jnp.int32)]).reshape(-1, CHUNK)

    ngrid = (-(-n // BR),)
    row_spec = pl.BlockSpec((BR, D), lambda i: (i, 0))
    col_spec = pl.BlockSpec((BR, 1), lambda i: (i, 0))
    s3_spec = pl.BlockSpec((NC, BR, HALF), lambda i: (0, i, 0))
    cw_spec = pl.BlockSpec(memory_space=pltpu.SMEM)

    scaled3 = pl.pallas_call(
        _prep_body,
        grid=ngrid,
        in_specs=[row_spec, col_spec, col_spec],
        out_specs=s3_spec,
        out_shape=jax.ShapeDtypeStruct((NC, n, HALF), jnp.float32),
    )(total, deg_new, deg_old)

    p3 = _sc_spmm(n, e_pad)(src, dst, scaled3)

    out = pl.pallas_call(
        _post_body,
        grid=ngrid,
        in_specs=[cw_spec, s3_spec, row_spec, row_spec, col_spec, col_spec],
        out_specs=row_spec,
        out_shape=jax.ShapeDtypeStruct((n, D), jnp.float32),
    )(conv_w, p3, total, old_emb1, deg_new, deg_old)

    return out


# speedup vs baseline: 1.1551x; 1.1551x over previous
import functools

import jax
import jax.numpy as jnp
from jax import lax
from jax.experimental import pallas as pl
from jax.experimental.pallas import tpu as pltpu
from jax.experimental.pallas import tpu_sc as plsc

D = 64
HALF = 32
N_PAD = 51200
CHUNK = 128
NS = 16
NC = 2
ZR = 128
BR = 512
IB = 16
NRING = 4


def _prep_body(tot_ref, dn_ref, do_ref, s3_ref):
    inv = 1.0 / (jnp.sqrt(dn_ref[...] + do_ref[...]) + 1e-9)
    scaled = inv * tot_ref[...]
    s3_ref[0] = scaled[:, :HALF]
    s3_ref[1] = scaled[:, HALF:]


def _post_body(cw_ref, p3_ref, tot_ref, old_ref, dn_ref, do_ref, out_ref):
    do = do_ref[...]
    inv = 1.0 / (jnp.sqrt(dn_ref[...] + do) + 1e-9)
    w10 = cw_ref[1, 0]
    w11 = cw_ref[1, 1]
    p = jnp.concatenate([p3_ref[0], p3_ref[1]], axis=1)
    last = (w10 * jnp.sqrt(do) * inv) * old_ref[...] + (w11 * inv) * p
    nrm = jnp.maximum(
        jnp.sqrt(jnp.sum(last * last, axis=-1, keepdims=True)), 1e-12)
    out_ref[...] = last / nrm + tot_ref[...]


def _sc_spmm(n_nodes, e_pad):
    e_per_tile = e_pad // NS
    n_chunks = e_per_tile // CHUNK
    nb = n_chunks // IB
    rows_per_tile = N_PAD // NS
    n_zr = rows_per_tile // ZR
    mesh = plsc.VectorSubcoreMesh(core_axis_name="c", subcore_axis_name="s")

    @functools.partial(
        pl.kernel,
        out_type=jax.ShapeDtypeStruct((NC, N_PAD, HALF), jnp.float32),
        mesh=mesh,
        compiler_params=pltpu.CompilerParams(use_tc_tiling_on_sc=False),
        scratch_types=[
            pltpu.VMEM((2, IB * CHUNK), jnp.int32),
            pltpu.VMEM((2, IB, CHUNK), jnp.int32),
            pltpu.VMEM((NRING, CHUNK, HALF), jnp.float32),
            pltpu.VMEM_SHARED((N_PAD, HALF), jnp.float32),
            pltpu.SemaphoreType.DMA,
            pltpu.SemaphoreType.DMA,
            pltpu.SemaphoreType.DMA,
        ],
    )
    def spmm(src_hbm, dst_hbm, s3_hbm, out_hbm,
             sibuf, dibuf, ring, acc, gsem, isem, ssem):
        c = lax.axis_index("c")
        s = lax.axis_index("s")
        s3c = s3_hbm.at[c]
        cbase = s * n_chunks
        ebase = s * e_per_tile

        def zrow(i, carry):
            z = jnp.zeros((16,), jnp.float32)
            ring[0, i, pl.ds(0, 16)] = z
            ring[0, i, pl.ds(16, 16)] = z
            return carry
        lax.fori_loop(0, CHUNK, zrow, 0)
        rbase = s * rows_per_tile

        def zcopy(i, carry):
            pltpu.sync_copy(ring.at[0], acc.at[pl.ds(rbase + i * ZR, ZR)])
            return carry
        lax.fori_loop(0, n_zr, zcopy, 0)
        plsc.subcore_barrier()

        ga = 2
        pltpu.sync_copy(src_hbm.at[pl.ds(ebase, IB * CHUNK)], sibuf.at[0])
        pltpu.sync_copy(dst_hbm.at[pl.ds(cbase, IB)], dibuf.at[0])
        sib0 = sibuf.at[0]
        for k in range(ga):
            pltpu.async_copy(
                s3c.at[sib0.at[pl.ds(k * CHUNK, CHUNK)]], ring.at[k], gsem)

        def block(b, carry):
            pb = lax.rem(b, 2)
            nxt_e = ebase + (b + 1) * IB * CHUNK
            nxt_c = cbase + (b + 1) * IB

            @pl.when(b < nb - 1)
            def _():
                pltpu.async_copy(
                    src_hbm.at[pl.ds(nxt_e, IB * CHUNK)], sibuf.at[1 - pb],
                    isem)
                pltpu.async_copy(
                    dst_hbm.at[pl.ds(nxt_c, IB)], dibuf.at[1 - pb], isem)

            sib = sibuf.at[pb]
            dib = dibuf.at[pb]
            sibn = sibuf.at[1 - pb]
            for k in range(IB):
                slot = k % NRING
                s2 = (k + ga) % NRING
                if k >= 2:
                    pltpu.make_async_copy(
                        ring.at[s2], acc.at[dib.at[k - 2]], ssem).wait()
                else:
                    @pl.when(b > 0)
                    def _():
                        pltpu.make_async_copy(
                            ring.at[s2], acc.at[dib.at[k]], ssem).wait()
                if k < IB - ga:
                    pltpu.async_copy(
                        s3c.at[sib.at[pl.ds((k + ga) * CHUNK, CHUNK)]],
                        ring.at[s2], gsem)
                else:
                    @pl.when(b < nb - 1)
                    def _():
                        if k == IB - ga:
                            pltpu.make_async_copy(
                                src_hbm.at[pl.ds(nxt_e, IB * CHUNK)],
                                sibuf.at[1 - pb], isem).wait()
                            pltpu.make_async_copy(
                                dst_hbm.at[pl.ds(nxt_c, IB)],
                                dibuf.at[1 - pb], isem).wait()
                        pltpu.async_copy(
                            s3c.at[sibn.at[
                                pl.ds((k - (IB - ga)) * CHUNK, CHUNK)]],
                            ring.at[s2], gsem)
                pltpu.make_async_copy(
                    s3c.at[sib.at[pl.ds(k * CHUNK, CHUNK)]], ring.at[slot],
                    gsem).wait()
                pltpu.async_copy(
                    ring.at[slot], acc.at[dib.at[k]], ssem, add=True)
            return carry
        lax.fori_loop(0, nb, block, 0)
        lk = (nb * IB - 2) % NRING
        pltpu.make_async_copy(
            ring.at[lk], acc.at[dibuf.at[0].at[0]], ssem).wait()
        pltpu.make_async_copy(
            ring.at[(lk + 1) % NRING], acc.at[dibuf.at[0].at[1]], ssem).wait()
        plsc.subcore_barrier()

        def wcopy(i, carry):
            r = rbase + i * ZR
            pltpu.sync_copy(acc.at[pl.ds(r, ZR)], ring.at[0])
            pltpu.sync_copy(ring.at[0], out_hbm.at[c].at[pl.ds(r, ZR)])
            return carry
        lax.fori_loop(0, n_zr, wcopy, 0)

    return spmm


def kernel(edge_index, now_user_degree, now_item_degree, old_user_degree,
           old_item_degree, old_emb0, old_emb1, user_table, item_table,
           conv_w):
    n = user_table.shape[0] + item_table.shape[0]
    e = edge_index.shape[1]
    e_pad = -(-e // (NS * CHUNK * IB)) * (NS * CHUNK * IB)

    total = jnp.concatenate([user_table, item_table], axis=0)
    deg_new = jnp.concatenate([now_user_degree, now_item_degree], axis=0)
    deg_old = jnp.concatenate([old_user_degree, old_item_degree], axis=0)
    ei = edge_index.astype(jnp.int32)
    pad_e = e_pad - e
    src = jnp.concatenate([ei[1], jnp.zeros((pad_e,), jnp.int32)])
    dst = jnp.concatenate(
        [ei[0], jnp.full((pad_e,), N_PAD - 1, jnp.int32)]).reshape(-1, CHUNK)

    ngrid = (-(-n // BR),)
    row_spec = pl.BlockSpec((BR, D), lambda i: (i, 0))
    col_spec = pl.BlockSpec((BR, 1), lambda i: (i, 0))
    s3_spec = pl.BlockSpec((NC, BR, HALF), lambda i: (0, i, 0))
    cw_spec = pl.BlockSpec(memory_space=pltpu.SMEM)

    scaled3 = pl.pallas_call(
        _prep_body,
        grid=ngrid,
        in_specs=[row_spec, col_spec, col_spec],
        out_specs=s3_spec,
        out_shape=jax.ShapeDtypeStruct((NC, n, HALF), jnp.float32),
    )(total, deg_new, deg_old)

    p3 = _sc_spmm(n, e_pad)(src, dst, scaled3)

    out = pl.pallas_call(
        _post_body,
        grid=ngrid,
        in_specs=[cw_spec, s3_spec, row_spec, row_spec, col_spec, col_spec],
        out_specs=row_spec,
        out_shape=jax.ShapeDtypeStruct((n, D), jnp.float32),
    )(conv_w, p3, total, old_emb1, deg_new, deg_old)

    return out
